# Initial kernel scaffold; baseline (speedup 1.0000x reference)
import jax
import jax.numpy as jnp
from jax.experimental import pallas as pl


def kernel(input, adj_matrix, W1, W2, W3):
    raise NotImplementedError("write your pallas kernel here")


# speedup vs baseline: 1.2132x; 1.2132x over previous
import jax
import jax.numpy as jnp
from jax.experimental import pallas as pl
from jax.experimental.pallas import tpu as pltpu

_N = 4096
_D = 64
_TM = 256
_T = _N // _TM


def _gcn3_kernel(x_ref, adj_ref, w1_ref, w2_ref, w3_ref, out_ref,
                 adj_bf, h1, h2, s_ref):
    p = pl.program_id(0)
    i = pl.program_id(1)
    rows = pl.ds(i * _TM, _TM)

    @pl.when(p == 0)
    def _phase0():
        @pl.when(i == 0)
        def _():
            s_ref[...] = jnp.dot(
                x_ref[...], w1_ref[...],
                preferred_element_type=jnp.float32).astype(jnp.bfloat16)

        a = adj_ref[...].astype(jnp.bfloat16)
        adj_bf[rows, :] = a
        h1[rows, :] = jnp.maximum(
            jnp.dot(a, s_ref[...], preferred_element_type=jnp.float32), 0.0)

    @pl.when(p == 1)
    def _phase1():
        @pl.when(i == 0)
        def _():
            s_ref[...] = jnp.dot(
                h1[...], w2_ref[...],
                preferred_element_type=jnp.float32).astype(jnp.bfloat16)

        h2[rows, :] = jnp.maximum(
            jnp.dot(adj_bf[rows, :], s_ref[...],
                    preferred_element_type=jnp.float32), 0.0)

    @pl.when(p == 2)
    def _phase2():
        @pl.when(i == 0)
        def _():
            s_ref[...] = jnp.dot(
                h2[...], w3_ref[...],
                preferred_element_type=jnp.float32).astype(jnp.bfloat16)

        out_ref[...] = jnp.maximum(
            jnp.dot(adj_bf[rows, :], s_ref[...],
                    preferred_element_type=jnp.float32), 0.0)


def kernel(input, adj_matrix, W1, W2, W3):
    return pl.pallas_call(
        _gcn3_kernel,
        grid=(3, _T),
        in_specs=[
            pl.BlockSpec((_N, _D), lambda p, i: (0, 0)),
            pl.BlockSpec((_TM, _N), lambda p, i: (jnp.where(p == 0, i, _T - 1), 0)),
            pl.BlockSpec((_D, _D), lambda p, i: (0, 0)),
            pl.BlockSpec((_D, _D), lambda p, i: (0, 0)),
            pl.BlockSpec((_D, _D), lambda p, i: (0, 0)),
        ],
        out_specs=pl.BlockSpec((_TM, _D), lambda p, i: (jnp.where(p == 2, i, 0), 0)),
        out_shape=jax.ShapeDtypeStruct((_N, _D), jnp.float32),
        scratch_shapes=[
            pltpu.VMEM((_N, _N), jnp.bfloat16),
            pltpu.VMEM((_N, _D), jnp.float32),
            pltpu.VMEM((_N, _D), jnp.float32),
            pltpu.VMEM((_N, _D), jnp.bfloat16),
        ],
        compiler_params=pltpu.CompilerParams(
            dimension_semantics=("arbitrary", "arbitrary")),
    )(input, adj_matrix, W1, W2, W3)
